# Initial kernel scaffold; baseline (speedup 1.0000x reference)
import jax
import jax.numpy as jnp
from jax.experimental import pallas as pl


def kernel(input_ids, word_table, pos_table, ln_gamma, ln_beta):
    raise NotImplementedError("write your pallas kernel here")


# speedup vs baseline: 6.3810x; 6.3810x over previous
import functools

import jax
import jax.numpy as jnp
from jax import lax
from jax.experimental import pallas as pl
from jax.experimental.pallas import tpu as pltpu
from jax.experimental.pallas import tpu_sc as plsc

B = 1024
S = 200
H = 128
N = B * S
NC, NS = 2, 16
NW = NC * NS
ROWS_PER_W = N // NW
CHUNK = 200
NCHUNK = ROWS_PER_W // CHUNK
LN_EPS = 1e-12

_mesh = plsc.VectorSubcoreMesh(core_axis_name="c", subcore_axis_name="s")


@functools.partial(
    pl.kernel,
    out_type=jax.ShapeDtypeStruct((N, H), jnp.float32),
    mesh=_mesh,
    scratch_types=[
        pltpu.VMEM((ROWS_PER_W,), jnp.int32),
        pltpu.VMEM((CHUNK, H), jnp.float32),
        pltpu.SemaphoreType.DMA,
    ],
)
def _sc_gather(ids_hbm, table_hbm, out_hbm, idx_v, buf, sem):
    wid = lax.axis_index("s") * NC + lax.axis_index("c")
    base = wid * ROWS_PER_W
    pltpu.sync_copy(ids_hbm.at[pl.ds(base, ROWS_PER_W)], idx_v)

    def body(g, carry):
        idx = idx_v.at[pl.ds(g * CHUNK, CHUNK)]
        pltpu.async_copy(table_hbm.at[idx], buf, sem).wait()
        pltpu.sync_copy(buf, out_hbm.at[pl.ds(base + g * CHUNK, CHUNK)])
        return carry

    lax.fori_loop(0, NCHUNK, body, 0)


def _ln_body(w_ref, pos_ref, g_ref, b_ref, o_ref):
    x = w_ref[...] + pos_ref[...][None]
    mean = jnp.mean(x, axis=-1, keepdims=True)
    xc = x - mean
    var = jnp.mean(xc * xc, axis=-1, keepdims=True)
    o_ref[...] = xc * lax.rsqrt(var + LN_EPS) * g_ref[...][None] + b_ref[...][None]


RB = 8

_tc_ln = pl.pallas_call(
    _ln_body,
    grid=(B // RB,),
    in_specs=[
        pl.BlockSpec((RB, S, H), lambda i: (i, 0, 0)),
        pl.BlockSpec((S, H), lambda i: (0, 0)),
        pl.BlockSpec((1, H), lambda i: (0, 0)),
        pl.BlockSpec((1, H), lambda i: (0, 0)),
    ],
    out_specs=pl.BlockSpec((RB, S, H), lambda i: (i, 0, 0)),
    out_shape=jax.ShapeDtypeStruct((B, S, H), jnp.float32),
)


@jax.jit
def kernel(input_ids, word_table, pos_table, ln_gamma, ln_beta):
    ids = input_ids.reshape(-1).astype(jnp.int32)
    wemb = _sc_gather(ids, word_table)
    pos = pos_table[:S]
    out = _tc_ln(
        wemb.reshape(B, S, H),
        pos,
        ln_gamma.reshape(1, H),
        ln_beta.reshape(1, H),
    )
    return out


# speedup vs baseline: 6.7763x; 1.0620x over previous
import functools

import jax
import jax.numpy as jnp
from jax import lax
from jax.experimental import pallas as pl
from jax.experimental.pallas import tpu as pltpu
from jax.experimental.pallas import tpu_sc as plsc

B = 1024
S = 200
H = 128
N = B * S
NC, NS = 2, 16
NW = NC * NS
ROWS_PER_W = N // NW
CHUNK = 400
NCHUNK = ROWS_PER_W // CHUNK
NPAIR = NCHUNK // 2
LN_EPS = 1e-12

_mesh = plsc.VectorSubcoreMesh(core_axis_name="c", subcore_axis_name="s")


@functools.partial(
    pl.kernel,
    out_type=jax.ShapeDtypeStruct((N, H), jnp.float32),
    mesh=_mesh,
    scratch_types=[
        pltpu.VMEM((ROWS_PER_W,), jnp.int32),
        pltpu.VMEM((CHUNK, H), jnp.float32),
        pltpu.VMEM((CHUNK, H), jnp.float32),
        pltpu.SemaphoreType.DMA,
        pltpu.SemaphoreType.DMA,
        pltpu.SemaphoreType.DMA,
        pltpu.SemaphoreType.DMA,
    ],
)
def _sc_gather(ids_hbm, table_hbm, out_hbm, idx_v, buf0, buf1,
               gsem0, gsem1, osem0, osem1):
    wid = lax.axis_index("s") * NC + lax.axis_index("c")
    base = wid * ROWS_PER_W
    pltpu.sync_copy(ids_hbm.at[pl.ds(base, ROWS_PER_W)], idx_v)

    def gather(g, buf, sem):
        pltpu.async_copy(table_hbm.at[idx_v.at[pl.ds(g * CHUNK, CHUNK)]],
                         buf, sem)

    def gather_wait(buf, sem):
        pltpu.make_async_copy(table_hbm.at[pl.ds(0, CHUNK)], buf, sem).wait()

    def out_start(g, buf, sem):
        pltpu.async_copy(buf, out_hbm.at[pl.ds(base + g * CHUNK, CHUNK)], sem)

    def out_wait(g, buf, sem):
        pltpu.make_async_copy(buf, out_hbm.at[pl.ds(base + g * CHUNK, CHUNK)],
                              sem).wait()

    gather(0, buf0, gsem0)

    def pair(i, carry):
        a = 2 * i
        b = a + 1
        gather(b, buf1, gsem1)
        gather_wait(buf0, gsem0)
        out_start(a, buf0, osem0)
        gather_wait(buf1, gsem1)
        out_start(b, buf1, osem1)
        out_wait(a, buf0, osem0)

        @pl.when(i + 1 < NPAIR)
        def _():
            gather(a + 2, buf0, gsem0)

        out_wait(b, buf1, osem1)
        return carry

    lax.fori_loop(0, NPAIR, pair, 0)


def _ln_body(w_ref, pos_ref, g_ref, b_ref, o_ref):
    x = w_ref[...] + pos_ref[...][None]
    mean = jnp.mean(x, axis=-1, keepdims=True)
    xc = x - mean
    var = jnp.mean(xc * xc, axis=-1, keepdims=True)
    o_ref[...] = xc * lax.rsqrt(var + LN_EPS) * g_ref[...][None] + b_ref[...][None]


RB = 8

_tc_ln = pl.pallas_call(
    _ln_body,
    grid=(B // RB,),
    in_specs=[
        pl.BlockSpec((RB, S, H), lambda i: (i, 0, 0)),
        pl.BlockSpec((S, H), lambda i: (0, 0)),
        pl.BlockSpec((1, H), lambda i: (0, 0)),
        pl.BlockSpec((1, H), lambda i: (0, 0)),
    ],
    out_specs=pl.BlockSpec((RB, S, H), lambda i: (i, 0, 0)),
    out_shape=jax.ShapeDtypeStruct((B, S, H), jnp.float32),
)


@jax.jit
def kernel(input_ids, word_table, pos_table, ln_gamma, ln_beta):
    ids = input_ids.reshape(-1).astype(jnp.int32)
    wemb = _sc_gather(ids, word_table)
    pos = pos_table[:S]
    out = _tc_ln(
        wemb.reshape(B, S, H),
        pos,
        ln_gamma.reshape(1, H),
        ln_beta.reshape(1, H),
    )
    return out
